# Initial kernel scaffold; baseline (speedup 1.0000x reference)
import jax
import jax.numpy as jnp
from jax.experimental import pallas as pl


def kernel(row_ptr, edge_scores):
    raise NotImplementedError("write your pallas kernel here")


# speedup vs baseline: 1084.7351x; 1084.7351x over previous
import jax
import jax.numpy as jnp
from jax import lax
from jax.experimental import pallas as pl
from jax.experimental.pallas import tpu as pltpu
from jax.experimental.pallas import tpu_sc as plsc

_K = 8
_N = 50000
_E = 1600000
_NC = 2
_NS = 16
_NW = _NC * _NS
_NPW = 1568
_NPAD = _NW * _NPW
_RP_PAD = _NPAD + 8
_C = 8192

_NEG_INF = float("-inf")


def _sload(ref, i):
    return ref[pl.ds(i, 16)][0]


def _sc_body(rp_hbm, es_hbm, vals_hbm, idx_hbm, rp_v, buf_v, vals_v, idx_v,
             sem):
    cid = lax.axis_index("c")
    sid = lax.axis_index("s")
    wid = (sid * _NC + cid).astype(jnp.int32)
    node0 = wid * _NPW

    pltpu.async_copy(rp_hbm.at[pl.ds(node0, _NPW + 8)],
                     rp_v.at[pl.ds(0, _NPW + 8)], sem).wait()

    lane = lax.iota(jnp.int32, 16)
    lane_lt8 = lane < _K
    neg_inf_v = jnp.full((16,), _NEG_INF, jnp.float32)
    neg_one_v = jnp.full((16,), -1, jnp.int32)

    e_lo = _sload(rp_v, 0)
    e_hi = _sload(rp_v, _NPW)
    base0 = (e_lo // 8) * 8
    nwin = jnp.maximum((e_hi - base0 + _C - 1) // _C, 1)

    def merge_steps(cur, stop, base, tv, ti):
        nsteps = (stop - cur + 15) // 16

        def step(j, st):
            tv3, ti3 = st
            c3 = cur + j * 16
            off = c3 - base
            v = buf_v[pl.ds(off, 16)]
            m = lane < (stop - c3)
            vm = jnp.where(m, v, neg_inf_v)
            im = jnp.where(m, c3 + lane, neg_one_v)
            sv, si = plsc.sort_key_val(vm, im, descending=True)
            rv = lax.rev(sv, (0,))
            ri = lax.rev(si, (0,))
            cv = jnp.where(lane_lt8, tv3, rv)
            ci = jnp.where(lane_lt8, ti3, ri)
            nv, ni = plsc.sort_key_val(cv, ci, descending=True)
            return (nv, ni)

        return lax.fori_loop(0, nsteps, step, (tv, ti))

    def window_body(k, st):
        n_cur, cur, tv, ti = st
        base = jnp.minimum(base0 + k * _C, _E - _C)
        pltpu.async_copy(es_hbm.at[pl.ds(base, _C)],
                         buf_v.at[pl.ds(0, _C)], sem).wait()
        wend = base + _C

        n_end = jnp.int32(0)
        for sbit in (1024, 512, 256, 128, 64, 32, 16, 8, 4, 2, 1):
            cand = n_end + sbit
            cc = jnp.minimum(cand, _NPW)
            ok = (cand <= _NPW) & (_sload(rp_v, cc) <= wend)
            n_end = jnp.where(ok, cand, n_end)

        def node_body(m, st2):
            cur2, tv2, ti2 = st2
            t = _sload(rp_v, m + 1)
            tv2, ti2 = merge_steps(cur2, t, base, tv2, ti2)
            pos = m * _K + lane
            plsc.store_scatter(vals_v, [pos], tv2, mask=lane_lt8)
            plsc.store_scatter(idx_v, [pos], ti2, mask=lane_lt8)
            return (t, neg_inf_v, neg_one_v)

        cur, tv, ti = lax.fori_loop(n_cur, n_end, node_body, (cur, tv, ti))

        pstop = jnp.minimum(wend, e_hi)
        tv, ti = merge_steps(cur, pstop, base, tv, ti)
        return (n_end, pstop, tv, ti)

    lax.fori_loop(
        0, nwin, window_body,
        (jnp.int32(0), e_lo, jnp.full((16,), _NEG_INF, jnp.float32),
         jnp.full((16,), -1, jnp.int32)))

    out0 = node0 * _K
    pltpu.async_copy(vals_v, vals_hbm.at[pl.ds(out0, _NPW * _K)], sem).wait()
    pltpu.async_copy(idx_v, idx_hbm.at[pl.ds(out0, _NPW * _K)], sem).wait()


@jax.jit
def _topk_sc(rp_pad, edge_scores):
    mesh = plsc.VectorSubcoreMesh(
        core_axis_name="c", subcore_axis_name="s",
        num_cores=_NC, num_subcores=_NS)
    fn = pl.kernel(
        _sc_body,
        out_type=[
            jax.ShapeDtypeStruct((_NPAD * _K,), jnp.float32),
            jax.ShapeDtypeStruct((_NPAD * _K,), jnp.int32),
        ],
        mesh=mesh,
        scratch_types=[
            pltpu.VMEM((_NPW + 24,), jnp.int32),
            pltpu.VMEM((_C + 16,), jnp.float32),
            pltpu.VMEM((_NPW * _K,), jnp.float32),
            pltpu.VMEM((_NPW * _K,), jnp.int32),
            pltpu.SemaphoreType.DMA,
        ],
        compiler_params=pltpu.CompilerParams(needs_layout_passes=False),
    )
    return fn(rp_pad, edge_scores)


def kernel(row_ptr, edge_scores):
    rp32 = row_ptr.astype(jnp.int32)
    pad = jnp.broadcast_to(rp32[-1:], (_RP_PAD - _N - 1,))
    rp_pad = jnp.concatenate([rp32, pad])
    vals, idx = _topk_sc(rp_pad, edge_scores)
    vals = vals.reshape(_NPAD, _K)[:_N]
    idx = idx.reshape(_NPAD, _K)[:_N].astype(jnp.int64)
    return vals, idx


# speedup vs baseline: 1098.2499x; 1.0125x over previous
import jax
import jax.numpy as jnp
from jax import lax
from jax.experimental import pallas as pl
from jax.experimental.pallas import tpu as pltpu
from jax.experimental.pallas import tpu_sc as plsc

_K = 8
_N = 50000
_E = 1600000
_NC = 2
_NS = 16
_NW = _NC * _NS
_NPW = 1568
_NPAD = _NW * _NPW
_RP_PAD = _NPAD + 8
_C = 8192
_SLOT = _C + 16

_NEG_INF = float("-inf")


def _sload(ref, i):
    return ref[pl.ds(i, 16)][0]


def _sc_body(rp_hbm, es_hbm, vals_hbm, idx_hbm, rp_v, buf_v, vals_v, idx_v,
             sem):
    cid = lax.axis_index("c")
    sid = lax.axis_index("s")
    wid = (sid * _NC + cid).astype(jnp.int32)
    node0 = wid * _NPW

    pltpu.async_copy(rp_hbm.at[pl.ds(node0, _NPW + 8)],
                     rp_v.at[pl.ds(0, _NPW + 8)], sem).wait()

    lane = lax.iota(jnp.int32, 16)
    lane_lt8 = lane < _K
    neg_inf_v = jnp.full((16,), _NEG_INF, jnp.float32)
    neg_one_v = jnp.full((16,), -1, jnp.int32)

    e_lo = _sload(rp_v, 0)
    e_hi = _sload(rp_v, _NPW)
    base0 = (e_lo // 8) * 8
    nwin = jnp.maximum((e_hi - base0 + _C - 1) // _C, 1)

    def win_base(k):
        return jnp.minimum(base0 + k * _C, _E - _C)

    pltpu.async_copy(es_hbm.at[pl.ds(win_base(0), _C)],
                     buf_v.at[pl.ds(0, _C)], sem)

    def merge_steps(cur, stop, delta, tv, ti, fresh):
        nsteps = (stop - cur + 15) // 16

        def block(c3, stop_):
            v = buf_v[pl.ds(c3 + delta, 16)]
            m = lane < (stop_ - c3)
            vm = jnp.where(m, v, neg_inf_v)
            im = jnp.where(m, c3 + lane, neg_one_v)
            return plsc.sort_key_val(vm, im, descending=True)

        def splice(sv, si, tv_, ti_):
            rv = lax.rev(sv, (0,))
            ri = lax.rev(si, (0,))
            cv = jnp.where(lane_lt8, tv_, rv)
            ci = jnp.where(lane_lt8, ti_, ri)
            nv, ni = plsc.sort_key_val(cv, ci, descending=True)
            return (nv, ni)

        sv0, si0 = block(cur, stop)

        def fresh_fn():
            return (sv0, si0)

        def merge_fn():
            return splice(sv0, si0, tv, ti)

        tv, ti = lax.cond(fresh, fresh_fn, merge_fn)

        def step(j, st):
            tv3, ti3 = st
            sv, si = block(cur + j * 16, stop)
            return splice(sv, si, tv3, ti3)

        tv, ti = lax.fori_loop(1, nsteps, step, (tv, ti))
        return tv, ti, fresh & (nsteps <= 0)

    def window_body(k, st):
        n_cur, cur, tv, ti, fresh = st
        base = win_base(k)
        slot = (k % 2) * _SLOT
        pltpu.make_async_copy(es_hbm.at[pl.ds(base, _C)],
                              buf_v.at[pl.ds(slot, _C)], sem).wait()

        @pl.when(k + 1 < nwin)
        def _():
            nslot = ((k + 1) % 2) * _SLOT
            pltpu.async_copy(es_hbm.at[pl.ds(win_base(k + 1), _C)],
                             buf_v.at[pl.ds(nslot, _C)], sem)

        wend = base + _C
        delta = slot - base

        n_end = jnp.int32(0)
        for sbit in (1024, 512, 256, 128, 64, 32, 16, 8, 4, 2, 1):
            cand = n_end + sbit
            cc = jnp.minimum(cand, _NPW)
            ok = (cand <= _NPW) & (_sload(rp_v, cc) <= wend)
            n_end = jnp.where(ok, cand, n_end)

        def node_body(m, st2):
            cur2, tv2, ti2, fresh2 = st2
            t = _sload(rp_v, m + 1)
            tv2, ti2, _ = merge_steps(cur2, t, delta, tv2, ti2, fresh2)
            vals_v[pl.ds(m * _K, 16)] = tv2
            idx_v[pl.ds(m * _K, 16)] = ti2
            return (t, neg_inf_v, neg_one_v, jnp.bool_(True))

        cur, tv, ti, fresh = lax.fori_loop(
            n_cur, n_end, node_body, (cur, tv, ti, fresh))

        pstop = jnp.minimum(wend, e_hi)
        tv, ti, fresh = merge_steps(cur, pstop, delta, tv, ti, fresh)
        return (n_end, pstop, tv, ti, fresh)

    lax.fori_loop(
        0, nwin, window_body,
        (jnp.int32(0), e_lo, jnp.full((16,), _NEG_INF, jnp.float32),
         jnp.full((16,), -1, jnp.int32), jnp.bool_(True)))

    out0 = node0 * _K
    pltpu.async_copy(vals_v.at[pl.ds(0, _NPW * _K)],
                     vals_hbm.at[pl.ds(out0, _NPW * _K)], sem).wait()
    pltpu.async_copy(idx_v.at[pl.ds(0, _NPW * _K)],
                     idx_hbm.at[pl.ds(out0, _NPW * _K)], sem).wait()


@jax.jit
def _topk_sc(rp_pad, edge_scores):
    mesh = plsc.VectorSubcoreMesh(
        core_axis_name="c", subcore_axis_name="s",
        num_cores=_NC, num_subcores=_NS)
    fn = pl.kernel(
        _sc_body,
        out_type=[
            jax.ShapeDtypeStruct((_NPAD * _K,), jnp.float32),
            jax.ShapeDtypeStruct((_NPAD * _K,), jnp.int32),
        ],
        mesh=mesh,
        scratch_types=[
            pltpu.VMEM((_NPW + 24,), jnp.int32),
            pltpu.VMEM((2 * _SLOT,), jnp.float32),
            pltpu.VMEM((_NPW * _K + 16,), jnp.float32),
            pltpu.VMEM((_NPW * _K + 16,), jnp.int32),
            pltpu.SemaphoreType.DMA,
        ],
        compiler_params=pltpu.CompilerParams(needs_layout_passes=False),
    )
    return fn(rp_pad, edge_scores)


def kernel(row_ptr, edge_scores):
    rp32 = row_ptr.astype(jnp.int32)
    pad = jnp.broadcast_to(rp32[-1:], (_RP_PAD - _N - 1,))
    rp_pad = jnp.concatenate([rp32, pad])
    vals, idx = _topk_sc(rp_pad, edge_scores)
    vals = vals.reshape(_NPAD, _K)[:_N]
    idx = idx.reshape(_NPAD, _K)[:_N].astype(jnp.int64)
    return vals, idx


# speedup vs baseline: 1441.5767x; 1.3126x over previous
import jax
import jax.numpy as jnp
from jax import lax
from jax.experimental import pallas as pl
from jax.experimental.pallas import tpu as pltpu
from jax.experimental.pallas import tpu_sc as plsc

_K = 8
_N = 50000
_E = 1600000
_NC = 2
_NS = 16
_NW = _NC * _NS
_NPW = 1568
_NPAD = _NW * _NPW
_RP_PAD = _NPAD + 8
_C = 8192
_SLOT = _C + 16

_NEG_INF = float("-inf")


def _sload(ref, i):
    return ref[pl.ds(i, 16)][0]


def _sc_body(rp_hbm, es_hbm, vals_hbm, idx_hbm, rp_v, buf_v, vals_v, idx_v,
             sem):
    cid = lax.axis_index("c")
    sid = lax.axis_index("s")
    wid = (sid * _NC + cid).astype(jnp.int32)
    node0 = wid * _NPW

    pltpu.async_copy(rp_hbm.at[pl.ds(node0, _NPW + 8)],
                     rp_v.at[pl.ds(0, _NPW + 8)], sem).wait()

    lane = lax.iota(jnp.int32, 16)
    lane_lt8 = lane < _K
    neg_inf_v = jnp.full((16,), _NEG_INF, jnp.float32)
    neg_one_v = jnp.full((16,), -1, jnp.int32)

    e_lo = _sload(rp_v, 0)
    e_hi = _sload(rp_v, _NPW)
    base0 = (e_lo // 8) * 8
    nwin = jnp.maximum((e_hi - base0 + _C - 1) // _C, 1)

    def win_base(k):
        return jnp.minimum(base0 + k * _C, _E - _C)

    pltpu.async_copy(es_hbm.at[pl.ds(win_base(0), _C)],
                     buf_v.at[pl.ds(0, _C)], sem)

    def merge_steps(cur, stop, delta, tv, ti, fresh):
        nsteps = (stop - cur + 15) // 16

        def block(c3, stop_):
            v = buf_v[pl.ds(c3 + delta, 16)]
            m = lane < (stop_ - c3)
            vm = jnp.where(m, v, neg_inf_v)
            im = jnp.where(m, c3 + lane, neg_one_v)
            return plsc.sort_key_val(vm, im, descending=True)

        def splice(sv, si, tv_, ti_):
            rv = lax.rev(sv, (0,))
            ri = lax.rev(si, (0,))
            cv = jnp.where(lane_lt8, tv_, rv)
            ci = jnp.where(lane_lt8, ti_, ri)
            nv, ni = plsc.sort_key_val(cv, ci, descending=True)
            return (nv, ni)

        sv0, si0 = block(cur, stop)

        def fresh_fn():
            return (sv0, si0)

        def merge_fn():
            return splice(sv0, si0, tv, ti)

        tv, ti = lax.cond(fresh, fresh_fn, merge_fn)

        def step(j, st):
            tv3, ti3 = st
            sv, si = block(cur + j * 16, stop)
            return splice(sv, si, tv3, ti3)

        tv, ti = lax.fori_loop(1, nsteps, step, (tv, ti))
        return tv, ti, fresh & (nsteps <= 0)

    def window_body(k, st):
        n_cur, cur, tv, ti, fresh = st
        base = win_base(k)
        slot = (k % 2) * _SLOT
        pltpu.make_async_copy(es_hbm.at[pl.ds(base, _C)],
                              buf_v.at[pl.ds(slot, _C)], sem).wait()

        @pl.when(k + 1 < nwin)
        def _():
            nslot = ((k + 1) % 2) * _SLOT
            pltpu.async_copy(es_hbm.at[pl.ds(win_base(k + 1), _C)],
                             buf_v.at[pl.ds(nslot, _C)], sem)

        wend = base + _C
        delta = slot - base

        n_end = jnp.int32(0)
        for sbit in (1024, 512, 256, 128, 64, 32, 16, 8, 4, 2, 1):
            cand = n_end + sbit
            cc = jnp.minimum(cand, _NPW)
            ok = (cand <= _NPW) & (_sload(rp_v, cc) <= wend)
            n_end = jnp.where(ok, cand, n_end)

        def node_body(m, st2):
            cur2, tv2, ti2, fresh2 = st2
            t = _sload(rp_v, m + 1)
            tv2, ti2, _ = merge_steps(cur2, t, delta, tv2, ti2, fresh2)
            vals_v[pl.ds(m * _K, 16)] = tv2
            half = lax.gather(
                ti2, (lane >> 1)[:, None],
                lax.GatherDimensionNumbers(
                    offset_dims=(), collapsed_slice_dims=(0,),
                    start_index_map=(0,)),
                (1,), mode=lax.GatherScatterMode.PROMISE_IN_BOUNDS)
            inter = jnp.where((lane & 1) == 0, half, half >> 31)
            idx_v[pl.ds(m * 2 * _K, 16)] = inter
            return (t, neg_inf_v, neg_one_v, jnp.bool_(True))

        cur, tv, ti, fresh = lax.fori_loop(
            n_cur, n_end, node_body, (cur, tv, ti, fresh))

        pstop = jnp.minimum(wend, e_hi)
        tv, ti, fresh = merge_steps(cur, pstop, delta, tv, ti, fresh)
        return (n_end, pstop, tv, ti, fresh)

    lax.fori_loop(
        0, nwin, window_body,
        (jnp.int32(0), e_lo, jnp.full((16,), _NEG_INF, jnp.float32),
         jnp.full((16,), -1, jnp.int32), jnp.bool_(True)))

    out0 = node0 * _K
    pltpu.async_copy(vals_v.at[pl.ds(0, _NPW * _K)],
                     vals_hbm.at[pl.ds(out0, _NPW * _K)], sem).wait()
    pltpu.async_copy(idx_v.at[pl.ds(0, _NPW * 2 * _K)],
                     idx_hbm.at[pl.ds(2 * out0, _NPW * 2 * _K)], sem).wait()


@jax.jit
def _topk_sc(rp_pad, edge_scores):
    mesh = plsc.VectorSubcoreMesh(
        core_axis_name="c", subcore_axis_name="s",
        num_cores=_NC, num_subcores=_NS)
    fn = pl.kernel(
        _sc_body,
        out_type=[
            jax.ShapeDtypeStruct((_NPAD * _K,), jnp.float32),
            jax.ShapeDtypeStruct((_NPAD * _K * 2,), jnp.int32),
        ],
        mesh=mesh,
        scratch_types=[
            pltpu.VMEM((_NPW + 24,), jnp.int32),
            pltpu.VMEM((2 * _SLOT,), jnp.float32),
            pltpu.VMEM((_NPW * _K + 16,), jnp.float32),
            pltpu.VMEM((_NPW * _K * 2,), jnp.int32),
            pltpu.SemaphoreType.DMA,
        ],
        compiler_params=pltpu.CompilerParams(needs_layout_passes=False),
    )
    return fn(rp_pad, edge_scores)


def kernel(row_ptr, edge_scores):
    rp32 = row_ptr.astype(jnp.int32)
    pad = jnp.broadcast_to(rp32[-1:], (_RP_PAD - _N - 1,))
    rp_pad = jnp.concatenate([rp32, pad])
    vals, idx32 = _topk_sc(rp_pad, edge_scores)
    vals = vals.reshape(_NPAD, _K)[:_N]
    idx = lax.bitcast_convert_type(
        idx32.reshape(_NPAD, _K, 2), jnp.int64)[:_N]
    return vals, idx


# speedup vs baseline: 4575.7371x; 3.1741x over previous
import jax
import jax.numpy as jnp
from jax import lax
from jax.experimental import pallas as pl
from jax.experimental.pallas import tpu as pltpu
from jax.experimental.pallas import tpu_sc as plsc

_K = 8
_N = 50000
_E = 1600000
_NC = 2
_NS = 16
_NW = _NC * _NS
_NPW = 1664
_NPAD = _NW * _NPW
_RP_PAD = _NPAD + 8
_C = 8192
_SLOT = _C + 16

_NEG_INF = float("-inf")


def _sload(ref, i):
    return ref[pl.ds(i, 16)][0]


def _sc_body(rp_hbm, es_hbm, vals_hbm, idx_hbm, rp_v, buf_v, vals_v, idx_v,
             sem):
    cid = lax.axis_index("c")
    sid = lax.axis_index("s")
    wid = (sid * _NC + cid).astype(jnp.int32)
    node0 = wid * _NPW

    pltpu.async_copy(rp_hbm.at[pl.ds(node0, _NPW + 8)],
                     rp_v.at[pl.ds(0, _NPW + 8)], sem).wait()

    lane = lax.iota(jnp.int32, 16)
    lane_lt8 = lane < _K
    neg_inf_v = jnp.full((16,), _NEG_INF, jnp.float32)
    neg_one_v = jnp.full((16,), -1, jnp.int32)

    e_lo = _sload(rp_v, 0)
    e_hi = _sload(rp_v, _NPW)
    base0 = (e_lo // 8) * 8
    nwin = jnp.maximum((e_hi - base0 + _C - 1) // _C, 1)

    def win_base(k):
        return jnp.minimum(base0 + k * _C, _E - _C)

    pltpu.async_copy(es_hbm.at[pl.ds(win_base(0), _C)],
                     buf_v.at[pl.ds(0, _C)], sem)

    def merge_steps(cur, stop, delta, tv, ti, fresh):
        nsteps = (stop - cur + 15) // 16

        def block(c3, stop_):
            v = buf_v[pl.ds(c3 + delta, 16)]
            m = lane < (stop_ - c3)
            vm = jnp.where(m, v, neg_inf_v)
            im = jnp.where(m, c3 + lane, neg_one_v)
            return plsc.sort_key_val(vm, im, descending=True)

        def splice(sv, si, tv_, ti_):
            rv = lax.rev(sv, (0,))
            ri = lax.rev(si, (0,))
            cv = jnp.where(lane_lt8, tv_, rv)
            ci = jnp.where(lane_lt8, ti_, ri)
            nv, ni = plsc.sort_key_val(cv, ci, descending=True)
            return (nv, ni)

        sv0, si0 = block(cur, stop)

        def fresh_fn():
            return (sv0, si0)

        def merge_fn():
            return splice(sv0, si0, tv, ti)

        tv, ti = lax.cond(fresh, fresh_fn, merge_fn)

        def step(j, st):
            tv3, ti3 = st
            sv, si = block(cur + j * 16, stop)
            return splice(sv, si, tv3, ti3)

        tv, ti = lax.fori_loop(1, nsteps, step, (tv, ti))
        return tv, ti, fresh & (nsteps <= 0)

    def window_body(k, st):
        n_cur, cur, tv, ti, fresh = st
        base = win_base(k)
        slot = (k % 2) * _SLOT
        pltpu.make_async_copy(es_hbm.at[pl.ds(base, _C)],
                              buf_v.at[pl.ds(slot, _C)], sem).wait()

        @pl.when(k + 1 < nwin)
        def _():
            nslot = ((k + 1) % 2) * _SLOT
            pltpu.async_copy(es_hbm.at[pl.ds(win_base(k + 1), _C)],
                             buf_v.at[pl.ds(nslot, _C)], sem)

        wend = base + _C
        delta = slot - base

        n_end = jnp.int32(0)
        for sbit in (1024, 512, 256, 128, 64, 32, 16, 8, 4, 2, 1):
            cand = n_end + sbit
            cc = jnp.minimum(cand, _NPW)
            ok = (cand <= _NPW) & (_sload(rp_v, cc) <= wend)
            n_end = jnp.where(ok, cand, n_end)

        def node_body(m, st2):
            cur2, tv2, ti2, fresh2 = st2
            t = _sload(rp_v, m + 1)
            tv2, ti2, _ = merge_steps(cur2, t, delta, tv2, ti2, fresh2)
            col = jnp.full((16,), m, jnp.int32)
            plsc.store_scatter(vals_v, [lane, col], tv2, mask=lane_lt8)
            plsc.store_scatter(idx_v, [lane, col], ti2, mask=lane_lt8)
            return (t, neg_inf_v, neg_one_v, jnp.bool_(True))

        cur, tv, ti, fresh = lax.fori_loop(
            n_cur, n_end, node_body, (cur, tv, ti, fresh))

        pstop = jnp.minimum(wend, e_hi)
        tv, ti, fresh = merge_steps(cur, pstop, delta, tv, ti, fresh)
        return (n_end, pstop, tv, ti, fresh)

    lax.fori_loop(
        0, nwin, window_body,
        (jnp.int32(0), e_lo, jnp.full((16,), _NEG_INF, jnp.float32),
         jnp.full((16,), -1, jnp.int32), jnp.bool_(True)))

    pltpu.async_copy(vals_v, vals_hbm.at[:, pl.ds(node0, _NPW)], sem).wait()
    pltpu.async_copy(idx_v, idx_hbm.at[:, pl.ds(node0, _NPW)], sem).wait()


@jax.jit
def _topk_sc(rp_pad, edge_scores):
    mesh = plsc.VectorSubcoreMesh(
        core_axis_name="c", subcore_axis_name="s",
        num_cores=_NC, num_subcores=_NS)
    fn = pl.kernel(
        _sc_body,
        out_type=[
            jax.ShapeDtypeStruct((_K, _NPAD), jnp.float32),
            jax.ShapeDtypeStruct((_K, _NPAD), jnp.int32),
        ],
        mesh=mesh,
        scratch_types=[
            pltpu.VMEM((_NPW + 24,), jnp.int32),
            pltpu.VMEM((2 * _SLOT,), jnp.float32),
            pltpu.VMEM((_K, _NPW), jnp.float32),
            pltpu.VMEM((_K, _NPW), jnp.int32),
            pltpu.SemaphoreType.DMA,
        ],
        compiler_params=pltpu.CompilerParams(needs_layout_passes=False),
    )
    return fn(rp_pad, edge_scores)


def kernel(row_ptr, edge_scores):
    rp32 = row_ptr.astype(jnp.int32)
    pad = jnp.broadcast_to(rp32[-1:], (_RP_PAD - _N - 1,))
    rp_pad = jnp.concatenate([rp32, pad])
    vals_t, idx_t = _topk_sc(rp_pad, edge_scores)
    vals = vals_t.T[:_N]
    idx = idx_t.T.astype(jnp.int64)[:_N]
    return vals, idx


# speedup vs baseline: 5166.2683x; 1.1291x over previous
import jax
import jax.numpy as jnp
from jax import lax
from jax.experimental import pallas as pl
from jax.experimental.pallas import tpu as pltpu
from jax.experimental.pallas import tpu_sc as plsc

_K = 8
_N = 50000
_E = 1600000
_NC = 2
_NS = 16
_NW = _NC * _NS
_NPW = 1664
_NPAD = _NW * _NPW
_RP_PAD = _NPAD + 8
_C = 8192
_SLOT = _C + 16

_NEG_INF = float("-inf")


def _sload(ref, i):
    return ref[pl.ds(i, 16)][0]


def _sc_body(rp_hbm, es_hbm, vals_hbm, idx_hbm, rp_v, buf_v, vals_v, idx_v,
             sem):
    cid = lax.axis_index("c")
    sid = lax.axis_index("s")
    wid = (sid * _NC + cid).astype(jnp.int32)
    node0 = wid * _NPW

    pltpu.async_copy(rp_hbm.at[pl.ds(node0, _NPW + 8)],
                     rp_v.at[pl.ds(0, _NPW + 8)], sem).wait()

    lane = lax.iota(jnp.int32, 16)
    lane_lt8 = lane < _K
    neg_inf_v = jnp.full((16,), _NEG_INF, jnp.float32)
    neg_one_v = jnp.full((16,), -1, jnp.int32)
    ones_b = jnp.ones((16,), jnp.bool_)

    e_lo = _sload(rp_v, 0)
    e_hi = _sload(rp_v, _NPW)
    base0 = (e_lo // 8) * 8
    nwin = jnp.maximum((e_hi - base0 + _C - 1) // _C, 1)

    def win_base(k):
        return jnp.minimum(base0 + k * _C, _E - _C)

    pltpu.async_copy(es_hbm.at[pl.ds(win_base(0), _C)],
                     buf_v.at[pl.ds(0, _C)], sem)

    def block(c3, stop_, delta):
        v = buf_v[pl.ds(c3 + delta, 16)]
        m = lane < (stop_ - c3)
        vm = jnp.where(m, v, neg_inf_v)
        im = jnp.where(m, c3 + lane, neg_one_v)
        return plsc.sort_key_val(vm, im, descending=True)

    def splice(sv, si, tv_, ti_):
        rv = lax.rev(sv, (0,))
        ri = lax.rev(si, (0,))
        cv = jnp.where(lane_lt8, tv_, rv)
        ci = jnp.where(lane_lt8, ti_, ri)
        nv, ni = plsc.sort_key_val(cv, ci, descending=True)
        return (nv, ni)

    def finalize(m, tv_, ti_, mask):
        col = jnp.full((16,), m, jnp.int32)
        plsc.store_scatter(vals_v, [lane, col], tv_, mask=mask)
        plsc.store_scatter(idx_v, [lane, col], ti_, mask=mask)

    def merge_steps(cur, stop, delta, tv, ti, fresh):
        nsteps = (stop - cur + 15) // 16

        sv0, si0 = block(cur, stop, delta)

        def fresh_fn():
            return (sv0, si0)

        def merge_fn():
            return splice(sv0, si0, tv, ti)

        tv, ti = lax.cond(fresh, fresh_fn, merge_fn)

        def step(j, st):
            sv, si = block(cur + j * 16, stop, delta)
            return splice(sv, si, *st)

        tv, ti = lax.fori_loop(1, nsteps, step, (tv, ti))
        return tv, ti, fresh & (nsteps <= 0)

    def window_body(k, st):
        n_cur, cur, tv, ti, fresh = st
        base = win_base(k)
        slot = (k % 2) * _SLOT
        pltpu.make_async_copy(es_hbm.at[pl.ds(base, _C)],
                              buf_v.at[pl.ds(slot, _C)], sem).wait()

        @pl.when(k + 1 < nwin)
        def _():
            nslot = ((k + 1) % 2) * _SLOT
            pltpu.async_copy(es_hbm.at[pl.ds(win_base(k + 1), _C)],
                             buf_v.at[pl.ds(nslot, _C)], sem)

        wend = base + _C
        delta = slot - base

        n_end = jnp.int32(0)
        for sbit in (1024, 512, 256, 128, 64, 32, 16, 8, 4, 2, 1):
            cand = n_end + sbit
            cc = jnp.minimum(cand, _NPW)
            ok = (cand <= _NPW) & (_sload(rp_v, cc) <= wend)
            n_end = jnp.where(ok, cand, n_end)

        def carry_fn():
            t = _sload(rp_v, n_cur + 1)
            stop = jnp.minimum(t, wend)
            nsteps = (stop - cur + 15) // 16
            sv0, si0 = block(cur, stop, delta)
            tv1, ti1 = splice(sv0, si0, tv, ti)

            def stp(j, s):
                sv, si = block(cur + j * 16, stop, delta)
                return splice(sv, si, *s)

            tv1, ti1 = lax.fori_loop(1, nsteps, stp, (tv1, ti1))
            done = t <= wend
            finalize(n_cur, tv1, ti1, done & lane_lt8)
            dv = done & ones_b
            tvn = jnp.where(dv, neg_inf_v, tv1)
            tin = jnp.where(dv, neg_one_v, ti1)
            return (n_cur + done.astype(jnp.int32), stop, tvn, tin, done)

        def fresh_pass():
            return (n_cur, cur, tv, ti, jnp.bool_(True))

        n_cur2, cur2, tv2, ti2, fresh2 = lax.cond(fresh, fresh_pass, carry_fn)

        def node_body(m, cur3):
            t = _sload(rp_v, m + 1)
            nsteps = (t - cur3 + 15) // 16
            sv0, si0 = block(cur3, t, delta)

            def stp(j, s):
                sv, si = block(cur3 + j * 16, t, delta)
                return splice(sv, si, *s)

            tv3, ti3 = lax.fori_loop(1, nsteps, stp, (sv0, si0))
            finalize(m, tv3, ti3, lane_lt8)
            return t

        cur2 = lax.fori_loop(n_cur2, n_end, node_body, cur2)

        pstop = jnp.minimum(wend, e_hi)
        tv2, ti2, fresh2 = merge_steps(cur2, pstop, delta, tv2, ti2, fresh2)
        return (n_end, pstop, tv2, ti2, fresh2)

    lax.fori_loop(
        0, nwin, window_body,
        (jnp.int32(0), e_lo, jnp.full((16,), _NEG_INF, jnp.float32),
         jnp.full((16,), -1, jnp.int32), jnp.bool_(True)))

    pltpu.async_copy(vals_v, vals_hbm.at[:, pl.ds(node0, _NPW)], sem).wait()
    pltpu.async_copy(idx_v, idx_hbm.at[:, pl.ds(node0, _NPW)], sem).wait()


@jax.jit
def _topk_sc(rp_pad, edge_scores):
    mesh = plsc.VectorSubcoreMesh(
        core_axis_name="c", subcore_axis_name="s",
        num_cores=_NC, num_subcores=_NS)
    fn = pl.kernel(
        _sc_body,
        out_type=[
            jax.ShapeDtypeStruct((_K, _NPAD), jnp.float32),
            jax.ShapeDtypeStruct((_K, _NPAD), jnp.int32),
        ],
        mesh=mesh,
        scratch_types=[
            pltpu.VMEM((_NPW + 24,), jnp.int32),
            pltpu.VMEM((2 * _SLOT,), jnp.float32),
            pltpu.VMEM((_K, _NPW), jnp.float32),
            pltpu.VMEM((_K, _NPW), jnp.int32),
            pltpu.SemaphoreType.DMA,
        ],
        compiler_params=pltpu.CompilerParams(needs_layout_passes=False),
    )
    return fn(rp_pad, edge_scores)


def kernel(row_ptr, edge_scores):
    rp32 = row_ptr.astype(jnp.int32)
    pad = jnp.broadcast_to(rp32[-1:], (_RP_PAD - _N - 1,))
    rp_pad = jnp.concatenate([rp32, pad])
    vals_t, idx_t = _topk_sc(rp_pad, edge_scores)
    vals = vals_t.T[:_N]
    idx = idx_t.T.astype(jnp.int64)[:_N]
    return vals, idx
